# Initial kernel scaffold; baseline (speedup 1.0000x reference)
import jax
import jax.numpy as jnp
from jax.experimental import pallas as pl


def kernel(atomic_features, num_features, feature_index, crystal_index, params):
    raise NotImplementedError("write your pallas kernel here")


# speedup vs baseline: 2.3827x; 2.3827x over previous
import functools

import jax
import jax.numpy as jnp
from jax.experimental import pallas as pl
from jax.experimental.pallas import tpu as pltpu
from jax.experimental.pallas import tpu_sc as plsc

_D = 64
_M = 16
_NB = 1000
_CB = 400
_EPS = 1e-5


def _softplus(x):
    return jnp.maximum(x, 0.0) + jnp.log1p(jnp.exp(-jnp.abs(x)))


def _sc_gather(table, idx, window):
    k = idx.shape[0]
    d = table.shape[1]
    idx2 = idx.reshape(1, k)
    mesh = plsc.VectorSubcoreMesh(core_axis_name="core", subcore_axis_name="subcore")

    @functools.partial(
        pl.kernel,
        out_type=jax.ShapeDtypeStruct((k, d), table.dtype),
        mesh=mesh,
        compiler_params=pltpu.CompilerParams(use_tc_tiling_on_sc=False),
    )
    def gather_kernel(tbl_hbm, idx_hbm, out_hbm):
        def body(i_vmem, o_vmem):
            pltpu.sync_copy(tbl_hbm.at[i_vmem.at[0]], o_vmem)

        pltpu.emit_pipeline(
            body,
            grid=(k // window,),
            in_specs=[pl.BlockSpec((1, window), lambda i: (0, i))],
            out_specs=[pl.BlockSpec((window, d), lambda i: (i, 0))],
            core_axis_name=("core", "subcore"),
            dimension_semantics=(pltpu.PARALLEL,),
        )(idx_hbm, out_hbm)

    return gather_kernel(table, idx2)


def _embed(a, w, b):
    n, fin = a.shape
    d = w.shape[1]

    def body(a_ref, w_ref, b_ref, o_ref):
        o_ref[...] = (
            jnp.dot(a_ref[...], w_ref[...], preferred_element_type=jnp.float32)
            + b_ref[...]
        )

    return pl.pallas_call(
        body,
        grid=(n // _NB,),
        in_specs=[
            pl.BlockSpec((_NB, fin), lambda i: (i, 0)),
            pl.BlockSpec((fin, d), lambda i: (0, 0)),
            pl.BlockSpec((1, d), lambda i: (0, 0)),
        ],
        out_specs=pl.BlockSpec((_NB, d), lambda i: (i, 0)),
        out_shape=jax.ShapeDtypeStruct((n, d), jnp.float32),
        compiler_params=pltpu.CompilerParams(dimension_semantics=("parallel",)),
    )(a, w, b)


def _conv_stats(g3, ef3, x, ws, wn, we):
    n = x.shape[0]
    grid = n // _NB
    d2 = ws.shape[1]
    fe = ef3.shape[2]

    def body(g_ref, ef_ref, x_ref, ws_ref, wn_ref, we_ref, s_ref, q_ref):
        u = jnp.dot(x_ref[...], ws_ref[...], preferred_element_type=jnp.float32)
        s = jnp.zeros((1, d2), jnp.float32)
        q = jnp.zeros((1, d2), jnp.float32)
        for m in range(_M):
            pre = jnp.dot(g_ref[m], wn_ref[...], preferred_element_type=jnp.float32)
            pre = pre + jnp.dot(ef_ref[m], we_ref[...], preferred_element_type=jnp.float32)
            pre = pre + u
            s = s + jnp.sum(pre, axis=0, keepdims=True)
            q = q + jnp.sum(pre * pre, axis=0, keepdims=True)
        s_ref[0] = s
        q_ref[0] = q

    return pl.pallas_call(
        body,
        grid=(grid,),
        in_specs=[
            pl.BlockSpec((_M, _NB, _D), lambda i: (0, i, 0)),
            pl.BlockSpec((_M, _NB, fe), lambda i: (0, i, 0)),
            pl.BlockSpec((_NB, _D), lambda i: (i, 0)),
            pl.BlockSpec((_D, d2), lambda i: (0, 0)),
            pl.BlockSpec((_D, d2), lambda i: (0, 0)),
            pl.BlockSpec((fe, d2), lambda i: (0, 0)),
        ],
        out_specs=[
            pl.BlockSpec((1, 1, d2), lambda i: (i, 0, 0)),
            pl.BlockSpec((1, 1, d2), lambda i: (i, 0, 0)),
        ],
        out_shape=[
            jax.ShapeDtypeStruct((grid, 1, d2), jnp.float32),
            jax.ShapeDtypeStruct((grid, 1, d2), jnp.float32),
        ],
        compiler_params=pltpu.CompilerParams(dimension_semantics=("parallel",)),
    )(g3, ef3, x, ws, wn, we)


def _conv_apply(g3, ef3, x, ws, wn, we, s1, t1):
    n = x.shape[0]
    grid = n // _NB
    d2 = ws.shape[1]
    fe = ef3.shape[2]

    def body(g_ref, ef_ref, x_ref, ws_ref, wn_ref, we_ref, s1_ref, t1_ref,
             sum_ref, s_ref, q_ref):
        u = jnp.dot(x_ref[...], ws_ref[...], preferred_element_type=jnp.float32)
        s1v = s1_ref[...]
        t1v = t1_ref[...]
        acc = jnp.zeros((_NB, _D), jnp.float32)
        for m in range(_M):
            pre = jnp.dot(g_ref[m], wn_ref[...], preferred_element_type=jnp.float32)
            pre = pre + jnp.dot(ef_ref[m], we_ref[...], preferred_element_type=jnp.float32)
            pre = pre + u
            z = pre * s1v + t1v
            f = z[:, :_D]
            c = z[:, _D:]
            acc = acc + jax.nn.sigmoid(f) * _softplus(c)
        sum_ref[...] = acc
        s_ref[0] = jnp.sum(acc, axis=0, keepdims=True)
        q_ref[0] = jnp.sum(acc * acc, axis=0, keepdims=True)

    return pl.pallas_call(
        body,
        grid=(grid,),
        in_specs=[
            pl.BlockSpec((_M, _NB, _D), lambda i: (0, i, 0)),
            pl.BlockSpec((_M, _NB, fe), lambda i: (0, i, 0)),
            pl.BlockSpec((_NB, _D), lambda i: (i, 0)),
            pl.BlockSpec((_D, d2), lambda i: (0, 0)),
            pl.BlockSpec((_D, d2), lambda i: (0, 0)),
            pl.BlockSpec((fe, d2), lambda i: (0, 0)),
            pl.BlockSpec((1, d2), lambda i: (0, 0)),
            pl.BlockSpec((1, d2), lambda i: (0, 0)),
        ],
        out_specs=[
            pl.BlockSpec((_NB, _D), lambda i: (i, 0)),
            pl.BlockSpec((1, 1, _D), lambda i: (i, 0, 0)),
            pl.BlockSpec((1, 1, _D), lambda i: (i, 0, 0)),
        ],
        out_shape=[
            jax.ShapeDtypeStruct((n, _D), jnp.float32),
            jax.ShapeDtypeStruct((grid, 1, _D), jnp.float32),
            jax.ShapeDtypeStruct((grid, 1, _D), jnp.float32),
        ],
        compiler_params=pltpu.CompilerParams(dimension_semantics=("parallel",)),
    )(g3, ef3, x, ws, wn, we, s1, t1)


def _residual(x, summed, s2, t2):
    n = x.shape[0]

    def body(x_ref, sm_ref, s2_ref, t2_ref, o_ref):
        o_ref[...] = _softplus(x_ref[...] + sm_ref[...] * s2_ref[...] + t2_ref[...])

    return pl.pallas_call(
        body,
        grid=(n // _NB,),
        in_specs=[
            pl.BlockSpec((_NB, _D), lambda i: (i, 0)),
            pl.BlockSpec((_NB, _D), lambda i: (i, 0)),
            pl.BlockSpec((1, _D), lambda i: (0, 0)),
            pl.BlockSpec((1, _D), lambda i: (0, 0)),
        ],
        out_specs=pl.BlockSpec((_NB, _D), lambda i: (i, 0)),
        out_shape=jax.ShapeDtypeStruct((n, _D), jnp.float32),
        compiler_params=pltpu.CompilerParams(dimension_semantics=("parallel",)),
    )(x, summed, s2, t2)


def _head(gc3, wfc, bfc, wh1, bh1, wh2, bh2):
    a, c, d = gc3.shape
    h = wfc.shape[1]

    def body(g_ref, wfc_ref, bfc_ref, wh1_ref, bh1_ref, wh2_ref, bh2_ref, o_ref):
        acc = jnp.zeros((_CB, d), jnp.float32)
        for m in range(a):
            acc = acc + g_ref[m]
        crys = _softplus(acc / jnp.float32(a))
        crys = jnp.dot(crys, wfc_ref[...], preferred_element_type=jnp.float32) + bfc_ref[...]
        crys = _softplus(crys)
        hh = _softplus(
            jnp.dot(crys, wh1_ref[...], preferred_element_type=jnp.float32) + bh1_ref[...]
        )
        o_ref[...] = (
            jnp.dot(hh, wh2_ref[...], preferred_element_type=jnp.float32) + bh2_ref[...]
        )

    return pl.pallas_call(
        body,
        grid=(c // _CB,),
        in_specs=[
            pl.BlockSpec((a, _CB, d), lambda i: (0, i, 0)),
            pl.BlockSpec((d, h), lambda i: (0, 0)),
            pl.BlockSpec((1, h), lambda i: (0, 0)),
            pl.BlockSpec((h, h), lambda i: (0, 0)),
            pl.BlockSpec((1, h), lambda i: (0, 0)),
            pl.BlockSpec((h, h), lambda i: (0, 0)),
            pl.BlockSpec((1, h), lambda i: (0, 0)),
        ],
        out_specs=pl.BlockSpec((_CB, h), lambda i: (i, 0)),
        out_shape=jax.ShapeDtypeStruct((c, h), jnp.float32),
        compiler_params=pltpu.CompilerParams(dimension_semantics=("parallel",)),
    )(gc3, wfc, bfc, wh1, bh1, wh2, bh2)


def kernel(atomic_features, num_features, feature_index, crystal_index, params):
    p = params
    n, m = feature_index.shape
    cnt = jnp.float32(n * m)

    x = _embed(atomic_features, p["W_emb"], p["b_emb"].reshape(1, -1))

    fi_t = feature_index.astype(jnp.int32).T.reshape(-1)
    ci_t = crystal_index.astype(jnp.int32).T.reshape(-1)
    ef3 = jnp.transpose(num_features, (1, 0, 2))

    for cp in p["convs"]:
        w = cp["W"]
        ws, wn, we = w[:_D], w[_D:2 * _D], w[2 * _D:]

        g3 = _sc_gather(x, fi_t, 128).reshape(m, n, _D)

        s_p, q_p = _conv_stats(g3, ef3, x, ws, wn, we)
        s = s_p.sum(axis=(0, 1))
        q = q_p.sum(axis=(0, 1))
        mu = s / cnt
        var = q / cnt - mu * mu
        inv = cp["g1"] / jnp.sqrt(var + _EPS)
        s1 = inv.reshape(1, -1)
        t1 = (cp["b1"] - mu * inv).reshape(1, -1)

        summed, s2_p, q2_p = _conv_apply(g3, ef3, x, ws, wn, we, s1, t1)
        s2 = s2_p.sum(axis=(0, 1))
        q2 = q2_p.sum(axis=(0, 1))
        mu2 = s2 / jnp.float32(n)
        var2 = q2 / jnp.float32(n) - mu2 * mu2
        inv2 = cp["g2"] / jnp.sqrt(var2 + _EPS)
        sc2 = inv2.reshape(1, -1)
        t2 = (cp["b2"] - mu2 * inv2).reshape(1, -1)

        x = _residual(x, summed, sc2, t2)

    c, a = crystal_index.shape
    gc3 = _sc_gather(x, ci_t, 80).reshape(a, c, _D)
    return _head(
        gc3,
        p["W_fc"], p["b_fc"].reshape(1, -1),
        p["W_h1"], p["b_h1"].reshape(1, -1),
        p["W_h2"], p["b_h2"].reshape(1, -1),
    )
